# baseline (speedup 1.0000x reference)
import functools

import jax
import jax.numpy as jnp
from jax import lax
from jax.experimental import pallas as pl
from jax.experimental.pallas import tpu as pltpu
from jax.experimental.pallas import tpu_sc as plsc

_B = 4096
_T = 200
_H = 64
_BT = _B * _T
_NW = 32
_C = 128

_PER = _BT // _NW
_SPER = _B // _NW


def _sc_gather_body(k0i, k1i, oi, s0i, s1i, s2i,
                    kt0, kt1, ot, st0, st1, st2,
                    tk0, tk1, to, ts0, ts1, ts2,
                    idx_v, rows_v, sem):
    wid = lax.axis_index("s") * 2 + lax.axis_index("c")

    def stream(idx_hbm, tab_hbm, out_hbm, base, nrows):
        @pl.loop(0, nrows // _C)
        def chunk(i):
            off = base + i * _C
            pltpu.sync_copy(idx_hbm.at[pl.ds(off, _C)], idx_v)
            pltpu.async_copy(tab_hbm.at[idx_v], rows_v, sem).wait()
            pltpu.sync_copy(rows_v, out_hbm.at[pl.ds(off, _C)])

    base = wid * _PER
    stream(k0i, kt0, tk0, base, _PER)
    stream(k1i, kt1, tk1, base, _PER)
    stream(oi, ot, to, base, _PER)
    sbase = wid * _SPER
    stream(s0i, st0, ts0, sbase, _SPER)
    stream(s1i, st1, ts1, sbase, _SPER)
    stream(s2i, st2, ts2, sbase, _SPER)


def _sc_gather(k0i, k1i, oi, s0i, s1i, s2i, kt0, kt1, ot, st0, st1, st2):
    mesh = plsc.VectorSubcoreMesh(core_axis_name="c", subcore_axis_name="s")
    row = lambda n: jax.ShapeDtypeStruct((n, _H), jnp.float32)
    f = pl.kernel(
        _sc_gather_body,
        out_type=(row(_BT), row(_BT), row(_BT), row(_B), row(_B), row(_B)),
        mesh=mesh,
        scratch_types=[
            pltpu.VMEM((_C,), jnp.int32),
            pltpu.VMEM((_C, _H), jnp.float32),
            pltpu.SemaphoreType.DMA,
        ],
        compiler_params=pltpu.CompilerParams(use_tc_tiling_on_sc=False),
    )
    return f(k0i, k1i, oi, s0i, s1i, s2i, kt0, kt1, ot, st0, st1, st2)


_NP = 512


def _asm_big_body(tk0, tk1, kc, wk, bk, to, oc, wo, bo, tg, te, tb,
                  outk, outo, outt):
    xk = jnp.concatenate([tk0[...], tk1[...], kc[...]], axis=1)
    outk[...] = jnp.dot(xk, wk[...], preferred_element_type=jnp.float32) + bk[...]
    xo = jnp.concatenate([to[...], oc[...]], axis=1)
    outo[...] = jnp.dot(xo, wo[...], preferred_element_type=jnp.float32) + bo[...]
    outt[...] = tg[...] * te[...] + tb[...]


def _asm_big(tk0, tk1, kc, wk, bk, to, oc, wo, bo, tg, te, tb):
    n = _NP
    grid = (_BT // n,)
    blk_row = pl.BlockSpec((n, _H), lambda i: (i, 0))
    blk_full = lambda r, c: pl.BlockSpec((r, c), lambda i: (0, 0))
    return pl.pallas_call(
        _asm_big_body,
        grid=grid,
        in_specs=[
            blk_row,
            blk_row,
            pl.BlockSpec((n, 8), lambda i: (i, 0)),
            blk_full(136, 640), blk_full(1, 640),
            blk_row,
            pl.BlockSpec((n, 8), lambda i: (i, 0)),
            blk_full(72, 576), blk_full(1, 576),
            pl.BlockSpec((n, 1), lambda i: (i, 0)),
            blk_full(1, _H), blk_full(1, _H),
        ],
        out_specs=[
            pl.BlockSpec((n, 640), lambda i: (i, 0)),
            pl.BlockSpec((n, 576), lambda i: (i, 0)),
            pl.BlockSpec((n, _H), lambda i: (i, 0)),
        ],
        out_shape=[
            jax.ShapeDtypeStruct((_BT, 640), jnp.float32),
            jax.ShapeDtypeStruct((_BT, 576), jnp.float32),
            jax.ShapeDtypeStruct((_BT, _H), jnp.float32),
        ],
    )(tk0, tk1, kc, wk, bk, to, oc, wo, bo, tg, te, tb)


def _asm_s_body(ts0, ts1, ts2, sc, ws, bs, outs):
    x = jnp.concatenate([ts0[...], ts1[...], ts2[...], sc[...]], axis=1)
    outs[...] = jnp.dot(x, ws[...], preferred_element_type=jnp.float32) + bs[...]


def _asm_s(ts0, ts1, ts2, sc, ws, bs):
    n = 512
    grid = (_B // n,)
    blk_row = pl.BlockSpec((n, _H), lambda i: (i, 0))
    return pl.pallas_call(
        _asm_s_body,
        grid=grid,
        in_specs=[
            blk_row, blk_row, blk_row,
            pl.BlockSpec((n, 4), lambda i: (i, 0)),
            pl.BlockSpec((196, 448), lambda i: (0, 0)),
            pl.BlockSpec((1, 448), lambda i: (0, 0)),
        ],
        out_specs=pl.BlockSpec((n, 448), lambda i: (i, 0)),
        out_shape=jax.ShapeDtypeStruct((_B, 448), jnp.float32),
    )(ts0, ts1, ts2, sc, ws, bs)


def kernel(s_cat, s_cont, k_cat, k_cont, o_cat, o_cont, target,
           s_cat_tables, k_cat_tables, o_cat_tables,
           s_cont_emb, s_cont_bias, k_cont_emb, k_cont_bias,
           o_cont_emb, o_cont_bias, tgt_emb, tgt_bias):
    k0i = k_cat[:, :, 0].reshape(_BT)
    k1i = k_cat[:, :, 1].reshape(_BT)
    oi = o_cat[:, :, 0].reshape(_BT)
    s0i = s_cat[:, 0, 0]
    s1i = s_cat[:, 0, 1]
    s2i = s_cat[:, 0, 2]

    tk0, tk1, to, ts0, ts1, ts2 = _sc_gather(
        k0i, k1i, oi, s0i, s1i, s2i,
        k_cat_tables[0], k_cat_tables[1], o_cat_tables[0],
        s_cat_tables[0], s_cat_tables[1], s_cat_tables[2])

    kc = k_cont.reshape(_BT, 8)
    oc = o_cont.reshape(_BT, 8)
    tg = target.reshape(_BT, 1)

    f32 = jnp.float32
    def cont_block(emb):
        m = emb.shape[0]
        return (jnp.eye(m, dtype=f32)[:, :, None]
                * emb[:, None, :]).reshape(m, m * _H)

    wk = jnp.concatenate([
        jnp.eye(128, 640, dtype=f32),
        jnp.concatenate([jnp.zeros((8, 128), f32), cont_block(k_cont_emb)], 1),
    ], 0)
    bk = jnp.concatenate([jnp.zeros((128,), f32),
                          k_cont_bias.reshape(512)])[None]
    wo = jnp.concatenate([
        jnp.eye(64, 576, dtype=f32),
        jnp.concatenate([jnp.zeros((8, 64), f32), cont_block(o_cont_emb)], 1),
    ], 0)
    bo = jnp.concatenate([jnp.zeros((64,), f32),
                          o_cont_bias.reshape(512)])[None]
    ws = jnp.concatenate([
        jnp.eye(192, 448, dtype=f32),
        jnp.concatenate([jnp.zeros((4, 192), f32), cont_block(s_cont_emb)], 1),
    ], 0)
    bs = jnp.concatenate([jnp.zeros((192,), f32),
                          s_cont_bias.reshape(256)])[None]

    outk, outo, outt = _asm_big(tk0, tk1, kc, wk, bk,
                                to, oc, wo, bo,
                                tg, tgt_emb, tgt_bias)
    outs = _asm_s(ts0, ts1, ts2, s_cont[:, 0, :], ws, bs)

    return (outs.reshape(_B, 7, _H),
            outk.reshape(_B, _T, 10, _H),
            outo.reshape(_B, _T, 9, _H),
            outt.reshape(_B, _T, 1, _H))


# speedup vs baseline: 1.4458x; 1.4458x over previous
import jax
import jax.numpy as jnp
from jax import lax
from jax.experimental import pallas as pl
from jax.experimental.pallas import tpu as pltpu
from jax.experimental.pallas import tpu_sc as plsc

_B = 4096
_T = 200
_H = 64
_BT = _B * _T
_NW = 32
_BPW = _B // _NW
_CK = 40

_i32 = jnp.int32
_f32 = jnp.float32


def _wid():
    return lax.axis_index("s") * 2 + lax.axis_index("c")


def _mesh():
    return plsc.VectorSubcoreMesh(core_axis_name="c", subcore_axis_name="s")


def _params():
    return pltpu.CompilerParams(use_tc_tiling_on_sc=False,
                                needs_layout_passes=False)


def _bcast16(ref, row, off):
    return plsc.load_gather(
        ref, [jnp.full((16,), row, _i32), jnp.full((16,), off, _i32)])


def _emb_regs(ev, bv, nfeat):
    e = [[ev[j, pl.ds(q * 16, 16)] for q in range(4)] for j in range(nfeat)]
    b = [[bv[j, pl.ds(q * 16, 16)] for q in range(4)] for j in range(nfeat)]
    return e, b


def _k_body(k0i, k1i, kcf, kt0, kt1, ke, kb, outk,
            idx0, idx1, st0, st1, cb, ch, ev, bv, sem_g, sem_o):
    wid = _wid()
    pltpu.sync_copy(ke, ev)
    pltpu.sync_copy(kb, bv)
    EV, BV = _emb_regs(ev, bv, 8)
    nchunks = _BPW * 5

    @pl.loop(0, nchunks // 2)
    def outer(g):
        for s in range(2):
            i = g * 2 + s
            b = wid * _BPW + i // 5
            t0 = (i % 5) * _CK
            p0 = b * _T + t0

            @pl.when(i >= 2)
            def _wait_prev():
                pltpu.make_async_copy(
                    ch.at[s], outk.at[0, pl.ds(0, _CK)], sem_o).wait()

            pltpu.sync_copy(k0i.at[pl.ds(p0, _CK)], idx0.at[s])
            pltpu.sync_copy(k1i.at[pl.ds(p0, _CK)], idx1.at[s])
            pltpu.sync_copy(kcf.at[pl.ds(p0 * 8, _CK * 8)], cb.at[s])
            d0 = pltpu.async_copy(kt0.at[idx0.at[s]], st0.at[s], sem_g)
            d1 = pltpu.async_copy(kt1.at[idx1.at[s]], st1.at[s], sem_g)
            d0.wait()
            d1.wait()

            @pl.loop(0, _CK // 2)
            def grp(g2):
                p = g2 * 2
                for pp in range(2):
                    for q in range(4):
                        sl = pl.ds(q * 16, 16)
                        ch[s, p + pp, 0, sl] = st0[s, p + pp, sl]
                        ch[s, p + pp, 1, sl] = st1[s, p + pp, sl]
                    for j in range(8):
                        bc = _bcast16(cb, s, (p + pp) * 8 + j)
                        for q in range(4):
                            ch[s, p + pp, 2 + j, pl.ds(q * 16, 16)] = (
                                bc * EV[j][q] + BV[j][q])

            pltpu.async_copy(ch.at[s], outk.at[b, pl.ds(t0, _CK)], sem_o)

    for s in range(2):
        pltpu.make_async_copy(
            ch.at[s], outk.at[0, pl.ds(0, _CK)], sem_o).wait()


def _k_pass(k0i, k1i, kcf, kt0, kt1, ke, kb):
    f = pl.kernel(
        _k_body,
        out_type=jax.ShapeDtypeStruct((_B, _T, 10, _H), _f32),
        mesh=_mesh(),
        scratch_types=[
            pltpu.VMEM((2, _CK), _i32),
            pltpu.VMEM((2, _CK), _i32),
            pltpu.VMEM((2, _CK, _H), _f32),
            pltpu.VMEM((2, _CK, _H), _f32),
            pltpu.VMEM((2, _CK * 8), _f32),
            pltpu.VMEM((2, _CK, 10, _H), _f32),
            pltpu.VMEM((8, _H), _f32),
            pltpu.VMEM((8, _H), _f32),
            pltpu.SemaphoreType.DMA,
            pltpu.SemaphoreType.DMA,
        ],
        compiler_params=_params(),
    )
    return f(k0i, k1i, kcf, kt0, kt1, ke, kb)


def _o_body(oi, ocf, ot, oe, ob, outo,
            idx0, st0, cb, ch, ev, bv, sem_g, sem_o):
    wid = _wid()
    pltpu.sync_copy(oe, ev)
    pltpu.sync_copy(ob, bv)
    EV, BV = _emb_regs(ev, bv, 8)
    nchunks = _BPW * 5

    @pl.loop(0, nchunks // 2)
    def outer(g):
        for s in range(2):
            i = g * 2 + s
            b = wid * _BPW + i // 5
            t0 = (i % 5) * _CK
            p0 = b * _T + t0

            @pl.when(i >= 2)
            def _wait_prev():
                pltpu.make_async_copy(
                    ch.at[s], outo.at[0, pl.ds(0, _CK)], sem_o).wait()

            pltpu.sync_copy(oi.at[pl.ds(p0, _CK)], idx0.at[s])
            pltpu.sync_copy(ocf.at[pl.ds(p0 * 8, _CK * 8)], cb.at[s])
            pltpu.async_copy(ot.at[idx0.at[s]], st0.at[s], sem_g).wait()

            @pl.loop(0, _CK // 2)
            def grp(g2):
                p = g2 * 2
                for pp in range(2):
                    for q in range(4):
                        sl = pl.ds(q * 16, 16)
                        ch[s, p + pp, 0, sl] = st0[s, p + pp, sl]
                    for j in range(8):
                        bc = _bcast16(cb, s, (p + pp) * 8 + j)
                        for q in range(4):
                            ch[s, p + pp, 1 + j, pl.ds(q * 16, 16)] = (
                                bc * EV[j][q] + BV[j][q])

            pltpu.async_copy(ch.at[s], outo.at[b, pl.ds(t0, _CK)], sem_o)

    for s in range(2):
        pltpu.make_async_copy(
            ch.at[s], outo.at[0, pl.ds(0, _CK)], sem_o).wait()


def _o_pass(oi, ocf, ot, oe, ob):
    f = pl.kernel(
        _o_body,
        out_type=jax.ShapeDtypeStruct((_B, _T, 9, _H), _f32),
        mesh=_mesh(),
        scratch_types=[
            pltpu.VMEM((2, _CK), _i32),
            pltpu.VMEM((2, _CK, _H), _f32),
            pltpu.VMEM((2, _CK * 8), _f32),
            pltpu.VMEM((2, _CK, 9, _H), _f32),
            pltpu.VMEM((8, _H), _f32),
            pltpu.VMEM((8, _H), _f32),
            pltpu.SemaphoreType.DMA,
            pltpu.SemaphoreType.DMA,
        ],
        compiler_params=_params(),
    )
    return f(oi, ocf, ot, oe, ob)


def _t_body(tvf, te, tb, outt, tvb, ch, ev, bv, sem_o):
    wid = _wid()
    pltpu.sync_copy(te, ev)
    pltpu.sync_copy(tb, bv)
    TE = [ev[0, pl.ds(q * 16, 16)] for q in range(4)]
    TB = [bv[0, pl.ds(q * 16, 16)] for q in range(4)]

    @pl.loop(0, _BPW // 2)
    def outer(g):
        for s in range(2):
            i = g * 2 + s
            b = wid * _BPW + i

            @pl.when(i >= 2)
            def _wait_prev():
                pltpu.make_async_copy(
                    ch.at[s, pl.ds(0, _T)], outt.at[0], sem_o).wait()

            pltpu.sync_copy(tvf.at[pl.ds(b * _T, _T)], tvb.at[s, pl.ds(0, _T)])

            @pl.loop(0, 13)
            def grp(g2):
                p = g2 * 16
                for pp in range(16):
                    bc = _bcast16(tvb, s, p + pp)
                    for q in range(4):
                        ch[s, p + pp, 0, pl.ds(q * 16, 16)] = (
                            bc * TE[q] + TB[q])

            pltpu.async_copy(ch.at[s, pl.ds(0, _T)], outt.at[b], sem_o)

    for s in range(2):
        pltpu.make_async_copy(
            ch.at[s, pl.ds(0, _T)], outt.at[0], sem_o).wait()


def _t_pass(tvf, te, tb):
    f = pl.kernel(
        _t_body,
        out_type=jax.ShapeDtypeStruct((_B, _T, 1, _H), _f32),
        mesh=_mesh(),
        scratch_types=[
            pltpu.VMEM((2, 208), _f32),
            pltpu.VMEM((2, 208, 1, _H), _f32),
            pltpu.VMEM((1, _H), _f32),
            pltpu.VMEM((1, _H), _f32),
            pltpu.SemaphoreType.DMA,
        ],
        compiler_params=_params(),
    )
    return f(tvf, te, tb)


def _s_body(s0i, s1i, s2i, scf, st0h, st1h, st2h, se, sb, outs,
            idx0, stg, cb, ch, ev, bv, sem_g):
    wid = _wid()
    pltpu.sync_copy(se, ev)
    pltpu.sync_copy(sb, bv)
    EV, BV = _emb_regs(ev, bv, 4)
    r0 = wid * _BPW

    for c, (ih, th) in enumerate(((s0i, st0h), (s1i, st1h), (s2i, st2h))):
        pltpu.sync_copy(ih.at[pl.ds(r0, _BPW)], idx0)
        pltpu.async_copy(th.at[idx0], stg, sem_g).wait()

        @pl.loop(0, _BPW)
        def cp(p):
            for q in range(4):
                sl = pl.ds(q * 16, 16)
                ch[p, c, sl] = stg[p, sl]

    pltpu.sync_copy(scf.at[pl.ds(r0 * 4, _BPW * 4)], cb.at[0])

    @pl.loop(0, _BPW // 4)
    def grp(g2):
        p = g2 * 4
        for pp in range(4):
            for j in range(4):
                bc = _bcast16(cb, 0, (p + pp) * 4 + j)
                for q in range(4):
                    ch[p + pp, 3 + j, pl.ds(q * 16, 16)] = (
                        bc * EV[j][q] + BV[j][q])

    pltpu.sync_copy(ch, outs.at[pl.ds(r0, _BPW)])


def _s_pass(s0i, s1i, s2i, scf, st0h, st1h, st2h, se, sb):
    f = pl.kernel(
        _s_body,
        out_type=jax.ShapeDtypeStruct((_B, 7, _H), _f32),
        mesh=_mesh(),
        scratch_types=[
            pltpu.VMEM((_BPW,), _i32),
            pltpu.VMEM((_BPW, _H), _f32),
            pltpu.VMEM((1, _BPW * 4), _f32),
            pltpu.VMEM((_BPW, 7, _H), _f32),
            pltpu.VMEM((4, _H), _f32),
            pltpu.VMEM((4, _H), _f32),
            pltpu.SemaphoreType.DMA,
        ],
        compiler_params=_params(),
    )
    return f(s0i, s1i, s2i, scf, st0h, st1h, st2h, se, sb)


def kernel(s_cat, s_cont, k_cat, k_cont, o_cat, o_cont, target,
           s_cat_tables, k_cat_tables, o_cat_tables,
           s_cont_emb, s_cont_bias, k_cont_emb, k_cont_bias,
           o_cont_emb, o_cont_bias, tgt_emb, tgt_bias):
    k0i = k_cat[:, :, 0].reshape(_BT)
    k1i = k_cat[:, :, 1].reshape(_BT)
    oi = o_cat[:, :, 0].reshape(_BT)
    s0i = s_cat[:, 0, 0]
    s1i = s_cat[:, 0, 1]
    s2i = s_cat[:, 0, 2]
    kcf = k_cont.reshape(_BT * 8)
    ocf = o_cont.reshape(_BT * 8)
    tvf = target.reshape(_BT)
    scf = s_cont.reshape(_B * 4)

    outk = _k_pass(k0i, k1i, kcf, k_cat_tables[0], k_cat_tables[1],
                   k_cont_emb, k_cont_bias)
    outo = _o_pass(oi, ocf, o_cat_tables[0], o_cont_emb, o_cont_bias)
    outt = _t_pass(tvf, tgt_emb, tgt_bias)
    outs = _s_pass(s0i, s1i, s2i, scf,
                   s_cat_tables[0], s_cat_tables[1], s_cat_tables[2],
                   s_cont_emb, s_cont_bias)

    return (outs, outk, outo, outt)
